# baseline (speedup 1.0000x reference)
import functools

import jax
import jax.numpy as jnp
from jax import lax
from jax.experimental import pallas as pl
from jax.experimental.pallas import tpu as pltpu
from jax.experimental.pallas import tpu_sc as plsc

N = 10000
D = 128
H = 64
NW = 32
CH = 128
K = 40
EPAD = NW * K * CH
NLANE = 16
HP = H + 8


def _pq_body(ne_ref, w1_ref, b1_ref, p_ref, q_ref):
    x = ne_ref[...]
    w1 = w1_ref[...]
    p_ref[...] = lax.dot_general(x, w1[:D, :], (((1,), (0,)), ((), ())),
                                 preferred_element_type=jnp.float32)
    q_ref[...] = lax.dot_general(x, w1[D:, :], (((1,), (0,)), ((), ())),
                                 preferred_element_type=jnp.float32) + b1_ref[...]


def _compute_pq(node_emb, W1, b1):
    blk = 1000
    return pl.pallas_call(
        _pq_body,
        grid=(N // blk,),
        in_specs=[
            pl.BlockSpec((blk, D), lambda i: (i, 0)),
            pl.BlockSpec((2 * D, H), lambda i: (0, 0)),
            pl.BlockSpec((1, H), lambda i: (0, 0)),
        ],
        out_specs=[
            pl.BlockSpec((blk, H), lambda i: (i, 0)),
            pl.BlockSpec((blk, H), lambda i: (i, 0)),
        ],
        out_shape=[
            jax.ShapeDtypeStruct((N, H), jnp.float32),
            jax.ShapeDtypeStruct((N, H), jnp.float32),
        ],
    )(node_emb, W1, b1.reshape(1, H))


NBUF = 4


def _sc_edge_body(p_hbm, q_hbm, src_hbm, dst_hbm, nz_hbm, ev_hbm, w2_hbm,
                  out_hbm, psum_hbm,
                  srcv, dstv, nzv, evv, outv, w2v, psv,
                  *bufs):
    pgs = bufs[0:NBUF]
    qgs = bufs[NBUF:2 * NBUF]
    sps = bufs[2 * NBUF:3 * NBUF]
    sqs = bufs[3 * NBUF:4 * NBUF]
    wid = lax.axis_index("s") * 2 + lax.axis_index("c")
    pltpu.sync_copy(src_hbm.at[wid], srcv)
    pltpu.sync_copy(dst_hbm.at[wid], dstv)
    pltpu.sync_copy(nz_hbm.at[wid], nzv)
    pltpu.sync_copy(ev_hbm.at[wid], evv)
    pltpu.sync_copy(w2_hbm, w2v)
    w2rows = [w2v[pl.ds(j * NLANE, NLANE)] for j in range(H // NLANE)]

    def issue(t, pg, qg, sp, sq):
        pltpu.make_async_copy(p_hbm.at[srcv.at[t]], pg, sp).start()
        pltpu.make_async_copy(q_hbm.at[dstv.at[t]], qg, sq).start()

    def wait(t, pg, qg, sp, sq):
        pltpu.make_async_copy(p_hbm.at[srcv.at[t]], pg, sp).wait()
        pltpu.make_async_copy(q_hbm.at[dstv.at[t]], qg, sq).wait()

    def compute(t, pg, qg, psum):
        def gbody(gg, psum):
            for half_g in range(2):
                g = gg * 2 + half_g
                rows = g * NLANE + lax.iota(jnp.int32, NLANE)
                accs = [jnp.zeros((NLANE,), jnp.float32) for _ in range(4)]
                for f in range(H):
                    fidx = jnp.full((NLANE,), f, jnp.int32)
                    pv = plsc.load_gather(pg, [rows, fidx])
                    qv = plsc.load_gather(qg, [rows, fidx])
                    w2f = w2rows[f // NLANE][f % NLANE]
                    accs[f % 4] = accs[f % 4] + jnp.maximum(pv + qv, 0.0) * w2f
                acc = (accs[0] + accs[1]) + (accs[2] + accs[3])
                nzg = nzv[t, pl.ds(g * NLANE, NLANE)]
                evg = evv[t, pl.ds(g * NLANE, NLANE)]
                aug = 1.0 / (1.0 + jnp.exp(-(acc + nzg)))
                outv[t, pl.ds(g * NLANE, NLANE)] = evg * aug
                psum = psum + aug
            return psum
        return lax.fori_loop(0, CH // NLANE // 2, gbody, psum)

    for b in range(NBUF - 1):
        issue(b, pgs[b], qgs[b], sps[b], sqs[b])

    def quad(i, psum):
        t0 = NBUF * i
        for b in range(NBUF):
            t = t0 + b
            wait(t, pgs[b], qgs[b], sps[b], sqs[b])
            psum = compute(t, pgs[b], qgs[b], psum)
            b2 = (b + NBUF - 1) % NBUF

            @pl.when(t + NBUF - 1 < K)
            def _():
                issue(t + NBUF - 1, pgs[b2], qgs[b2], sps[b2], sqs[b2])

        return psum

    psum = lax.fori_loop(0, K // NBUF, quad, jnp.zeros((NLANE,), jnp.float32))
    psv[...] = psum
    pltpu.sync_copy(outv, out_hbm.at[wid])
    pltpu.sync_copy(psv, psum_hbm.at[wid])


def _make_sc_call():
    mesh = plsc.VectorSubcoreMesh(core_axis_name="c", subcore_axis_name="s")
    return pl.kernel(
        _sc_edge_body,
        mesh=mesh,
        compiler_params=pltpu.CompilerParams(
            needs_layout_passes=False,
            use_tc_tiling_on_sc=False,
        ),
        out_type=[
            jax.ShapeDtypeStruct((NW, K, CH), jnp.float32),
            jax.ShapeDtypeStruct((NW, NLANE), jnp.float32),
        ],
        scratch_types=[
            pltpu.VMEM((K, CH), jnp.int32),
            pltpu.VMEM((K, CH), jnp.int32),
            pltpu.VMEM((K, CH), jnp.float32),
            pltpu.VMEM((K, CH), jnp.float32),
            pltpu.VMEM((K, CH), jnp.float32),
            pltpu.VMEM((H,), jnp.float32),
            pltpu.VMEM((NLANE,), jnp.float32),
            *[pltpu.VMEM((CH, HP), jnp.float32) for _ in range(2 * NBUF)],
            *[pltpu.SemaphoreType.DMA for _ in range(2 * NBUF)],
        ],
    )


def kernel(node_emb, edge_index, edge_vals, W1, b1, W2, b2):
    half = edge_index.shape[1] // 2
    src = edge_index[0, :half]
    dst = edge_index[1, :half]

    p, q = _compute_pq(node_emb, W1, b1)
    p = jnp.pad(p, ((0, 0), (0, HP - H)))
    q = jnp.pad(q, ((0, 0), (0, HP - H)))

    bias = 0.0 + 0.0001
    u = jax.random.uniform(jax.random.key(42), (half, 1), dtype=jnp.float32)
    eps = (bias - (1.0 - bias)) * u + (1.0 - bias)
    noise = (jnp.log(eps) - jnp.log(1.0 - eps)).squeeze(-1)
    nz = noise + b2[0]

    pad = EPAD - half
    srcp = jnp.concatenate([src, jnp.zeros((pad,), jnp.int32)]).reshape(NW, K, CH)
    dstp = jnp.concatenate([dst, jnp.zeros((pad,), jnp.int32)]).reshape(NW, K, CH)
    nzp = jnp.concatenate([nz, jnp.full((pad,), -1e30, jnp.float32)]).reshape(NW, K, CH)
    evp = jnp.concatenate([edge_vals[:half], jnp.zeros((pad,), jnp.float32)]).reshape(NW, K, CH)

    outp, psum = _make_sc_call()(p, q, srcp, dstp, nzp, evp, W2.reshape(H))

    new_vals = outp.reshape(-1)[:half]
    sym_inds = jnp.concatenate([jnp.stack([src, dst]), jnp.stack([dst, src])], axis=1)
    sym_vals = jnp.concatenate([new_vals, new_vals], axis=0)
    mean_edge_weight = jnp.sum(psum) / half
    return (sym_inds, sym_vals, mean_edge_weight)


# speedup vs baseline: 1.1415x; 1.1415x over previous
import functools

import jax
import jax.numpy as jnp
from jax import lax
from jax.experimental import pallas as pl
from jax.experimental.pallas import tpu as pltpu
from jax.experimental.pallas import tpu_sc as plsc

N = 10000
D = 128
H = 64
NW = 32
CH = 128
K = 40
EPAD = NW * K * CH
NLANE = 16
PAY = H // 2
HPW = PAY + 8


def _pq_body(ne_ref, w1_ref, b1_ref, p_ref, q_ref):
    x = ne_ref[...]
    w1 = w1_ref[...]
    p_ref[...] = lax.dot_general(x, w1[:D, :], (((1,), (0,)), ((), ())),
                                 preferred_element_type=jnp.float32)
    q_ref[...] = lax.dot_general(x, w1[D:, :], (((1,), (0,)), ((), ())),
                                 preferred_element_type=jnp.float32) + b1_ref[...]


def _compute_pq(node_emb, W1, b1):
    blk = 1000
    return pl.pallas_call(
        _pq_body,
        grid=(N // blk,),
        in_specs=[
            pl.BlockSpec((blk, D), lambda i: (i, 0)),
            pl.BlockSpec((2 * D, H), lambda i: (0, 0)),
            pl.BlockSpec((1, H), lambda i: (0, 0)),
        ],
        out_specs=[
            pl.BlockSpec((blk, H), lambda i: (i, 0)),
            pl.BlockSpec((blk, H), lambda i: (i, 0)),
        ],
        out_shape=[
            jax.ShapeDtypeStruct((N, H), jnp.float32),
            jax.ShapeDtypeStruct((N, H), jnp.float32),
        ],
    )(node_emb, W1, b1.reshape(1, H))


NBUF = 4


def _sc_edge_body(p_hbm, q_hbm, src_hbm, dst_hbm, nz_hbm, ev_hbm, w2_hbm,
                  out_hbm, psum_hbm,
                  srcv, dstv, nzv, evv, outv, w2v, psv,
                  *bufs):
    pgs = bufs[0:NBUF]
    qgs = bufs[NBUF:2 * NBUF]
    sps = bufs[2 * NBUF:3 * NBUF]
    sqs = bufs[3 * NBUF:4 * NBUF]
    wid = lax.axis_index("s") * 2 + lax.axis_index("c")
    pltpu.sync_copy(src_hbm.at[wid], srcv)
    pltpu.sync_copy(dst_hbm.at[wid], dstv)
    pltpu.sync_copy(nz_hbm.at[wid], nzv)
    pltpu.sync_copy(ev_hbm.at[wid], evv)
    pltpu.sync_copy(w2_hbm, w2v)
    w2rows = [w2v[pl.ds(j * NLANE, NLANE)] for j in range(H // NLANE)]

    def issue(t, pg, qg, sp, sq):
        pltpu.make_async_copy(p_hbm.at[srcv.at[t]], pg, sp).start()
        pltpu.make_async_copy(q_hbm.at[dstv.at[t]], qg, sq).start()

    def wait(t, pg, qg, sp, sq):
        pltpu.make_async_copy(p_hbm.at[srcv.at[t]], pg, sp).wait()
        pltpu.make_async_copy(q_hbm.at[dstv.at[t]], qg, sq).wait()

    def compute(t, pg, qg, psum):
        def gbody(gg, psum):
            for half_g in range(2):
                g = gg * 2 + half_g
                rows = g * NLANE + lax.iota(jnp.int32, NLANE)
                accs = [jnp.zeros((NLANE,), jnp.float32) for _ in range(4)]
                himask = jnp.full((NLANE,), -65536, jnp.int32)
                for k in range(PAY):
                    kidx = jnp.full((NLANE,), k, jnp.int32)
                    pk = plsc.load_gather(pg, [rows, kidx])
                    qk = plsc.load_gather(qg, [rows, kidx])
                    plo = lax.bitcast_convert_type(pk << 16, jnp.float32)
                    qlo = lax.bitcast_convert_type(qk << 16, jnp.float32)
                    phi = lax.bitcast_convert_type(pk & himask, jnp.float32)
                    qhi = lax.bitcast_convert_type(qk & himask, jnp.float32)
                    f0, f1 = 2 * k, 2 * k + 1
                    w2a = w2rows[f0 // NLANE][f0 % NLANE]
                    w2b = w2rows[f1 // NLANE][f1 % NLANE]
                    accs[f0 % 4] = accs[f0 % 4] + jnp.maximum(plo + qlo, 0.0) * w2a
                    accs[f1 % 4] = accs[f1 % 4] + jnp.maximum(phi + qhi, 0.0) * w2b
                acc = (accs[0] + accs[1]) + (accs[2] + accs[3])
                nzg = nzv[t, pl.ds(g * NLANE, NLANE)]
                evg = evv[t, pl.ds(g * NLANE, NLANE)]
                aug = 1.0 / (1.0 + jnp.exp(-(acc + nzg)))
                outv[t, pl.ds(g * NLANE, NLANE)] = evg * aug
                psum = psum + aug
            return psum
        return lax.fori_loop(0, CH // NLANE // 2, gbody, psum)

    for b in range(NBUF - 1):
        issue(b, pgs[b], qgs[b], sps[b], sqs[b])

    def quad(i, psum):
        t0 = NBUF * i
        for b in range(NBUF):
            t = t0 + b
            wait(t, pgs[b], qgs[b], sps[b], sqs[b])
            psum = compute(t, pgs[b], qgs[b], psum)
            b2 = (b + NBUF - 1) % NBUF

            @pl.when(t + NBUF - 1 < K)
            def _():
                issue(t + NBUF - 1, pgs[b2], qgs[b2], sps[b2], sqs[b2])

        return psum

    psum = lax.fori_loop(0, K // NBUF, quad, jnp.zeros((NLANE,), jnp.float32))
    psv[...] = psum
    pltpu.sync_copy(outv, out_hbm.at[wid])
    pltpu.sync_copy(psv, psum_hbm.at[wid])


def _make_sc_call():
    mesh = plsc.VectorSubcoreMesh(core_axis_name="c", subcore_axis_name="s")
    return pl.kernel(
        _sc_edge_body,
        mesh=mesh,
        compiler_params=pltpu.CompilerParams(
            needs_layout_passes=False,
            use_tc_tiling_on_sc=False,
        ),
        out_type=[
            jax.ShapeDtypeStruct((NW, K, CH), jnp.float32),
            jax.ShapeDtypeStruct((NW, NLANE), jnp.float32),
        ],
        scratch_types=[
            pltpu.VMEM((K, CH), jnp.int32),
            pltpu.VMEM((K, CH), jnp.int32),
            pltpu.VMEM((K, CH), jnp.float32),
            pltpu.VMEM((K, CH), jnp.float32),
            pltpu.VMEM((K, CH), jnp.float32),
            pltpu.VMEM((H,), jnp.float32),
            pltpu.VMEM((NLANE,), jnp.float32),
            *[pltpu.VMEM((CH, HPW), jnp.int32) for _ in range(2 * NBUF)],
            *[pltpu.SemaphoreType.DMA for _ in range(2 * NBUF)],
        ],
    )


def kernel(node_emb, edge_index, edge_vals, W1, b1, W2, b2):
    half = edge_index.shape[1] // 2
    src = edge_index[0, :half]
    dst = edge_index[1, :half]

    p, q = _compute_pq(node_emb, W1, b1)
    def _pack(t):
        ti = lax.bitcast_convert_type(
            t.astype(jnp.bfloat16).reshape(N, PAY, 2), jnp.int32)
        return jnp.pad(ti, ((0, 0), (0, HPW - PAY)))
    p = _pack(p)
    q = _pack(q)

    bias = 0.0 + 0.0001
    u = jax.random.uniform(jax.random.key(42), (half, 1), dtype=jnp.float32)
    eps = (bias - (1.0 - bias)) * u + (1.0 - bias)
    noise = (jnp.log(eps) - jnp.log(1.0 - eps)).squeeze(-1)
    nz = noise + b2[0]

    pad = EPAD - half
    srcp = jnp.concatenate([src, jnp.zeros((pad,), jnp.int32)]).reshape(NW, K, CH)
    dstp = jnp.concatenate([dst, jnp.zeros((pad,), jnp.int32)]).reshape(NW, K, CH)
    nzp = jnp.concatenate([nz, jnp.full((pad,), -1e30, jnp.float32)]).reshape(NW, K, CH)
    evp = jnp.concatenate([edge_vals[:half], jnp.zeros((pad,), jnp.float32)]).reshape(NW, K, CH)

    outp, psum = _make_sc_call()(p, q, srcp, dstp, nzp, evp, W2.reshape(H))

    new_vals = outp.reshape(-1)[:half]
    sym_inds = jnp.concatenate([jnp.stack([src, dst]), jnp.stack([dst, src])], axis=1)
    sym_vals = jnp.concatenate([new_vals, new_vals], axis=0)
    mean_edge_weight = jnp.sum(psum) / half
    return (sym_inds, sym_vals, mean_edge_weight)


# speedup vs baseline: 1.1473x; 1.0051x over previous
import functools

import jax
import jax.numpy as jnp
from jax import lax
from jax.experimental import pallas as pl
from jax.experimental.pallas import tpu as pltpu
from jax.experimental.pallas import tpu_sc as plsc

N = 10000
D = 128
H = 64
NW = 32
CH = 128
K = 40
EPAD = NW * K * CH
NLANE = 16
PAY = H // 2
HPW = PAY + 8


def _pq_body(ne_ref, w1_ref, b1_ref, p_ref, q_ref):
    x = ne_ref[...]
    w1 = w1_ref[...]
    p_ref[...] = lax.dot_general(x, w1[:D, :], (((1,), (0,)), ((), ())),
                                 preferred_element_type=jnp.float32)
    q_ref[...] = lax.dot_general(x, w1[D:, :], (((1,), (0,)), ((), ())),
                                 preferred_element_type=jnp.float32) + b1_ref[...]


def _compute_pq(node_emb, W1, b1):
    blk = 1000
    return pl.pallas_call(
        _pq_body,
        grid=(N // blk,),
        in_specs=[
            pl.BlockSpec((blk, D), lambda i: (i, 0)),
            pl.BlockSpec((2 * D, H), lambda i: (0, 0)),
            pl.BlockSpec((1, H), lambda i: (0, 0)),
        ],
        out_specs=[
            pl.BlockSpec((blk, H), lambda i: (i, 0)),
            pl.BlockSpec((blk, H), lambda i: (i, 0)),
        ],
        out_shape=[
            jax.ShapeDtypeStruct((N, H), jnp.float32),
            jax.ShapeDtypeStruct((N, H), jnp.float32),
        ],
    )(node_emb, W1, b1.reshape(1, H))


NBUF = 4


def _sc_edge_body(p_hbm, q_hbm, src_hbm, dst_hbm, nz_hbm, ev_hbm, w2_hbm,
                  out_hbm, psum_hbm,
                  srcv, dstv, nzv, evv, outv, w2v, psv,
                  *bufs):
    pgs = bufs[0:NBUF]
    qgs = bufs[NBUF:2 * NBUF]
    sps = bufs[2 * NBUF:3 * NBUF]
    sqs = bufs[3 * NBUF:4 * NBUF]
    wid = lax.axis_index("s") * 2 + lax.axis_index("c")
    pltpu.sync_copy(src_hbm.at[wid], srcv)
    pltpu.sync_copy(dst_hbm.at[wid], dstv)
    pltpu.sync_copy(nz_hbm.at[wid], nzv)
    pltpu.sync_copy(ev_hbm.at[wid], evv)
    pltpu.sync_copy(w2_hbm, w2v)
    w2rows = [w2v[pl.ds(j * NLANE, NLANE)] for j in range(H // NLANE)]

    def issue(t, pg, qg, sp, sq):
        pltpu.make_async_copy(p_hbm.at[srcv.at[t]], pg, sp).start()
        pltpu.make_async_copy(q_hbm.at[dstv.at[t]], qg, sq).start()

    def wait(t, pg, qg, sp, sq):
        pltpu.make_async_copy(p_hbm.at[srcv.at[t]], pg, sp).wait()
        pltpu.make_async_copy(q_hbm.at[dstv.at[t]], qg, sq).wait()

    def compute(t, pg, qg, psum):
        def gbody(gg, psum):
            for half_g in range(2):
                g = gg * 2 + half_g
                rows = g * NLANE + lax.iota(jnp.int32, NLANE)
                accs = [jnp.zeros((NLANE,), jnp.float32) for _ in range(4)]
                himask = jnp.full((NLANE,), -65536, jnp.int32)
                zero_b = jnp.zeros((2 * NLANE,), jnp.bfloat16)
                for k in range(PAY):
                    kidx = jnp.full((NLANE,), k, jnp.int32)
                    pk = plsc.load_gather(pg, [rows, kidx])
                    qk = plsc.load_gather(qg, [rows, kidx])
                    pb = plsc.bitcast(pk, jnp.bfloat16)
                    qb = plsc.bitcast(qk, jnp.bfloat16)
                    r = plsc.bitcast(jnp.maximum(pb + qb, zero_b), jnp.int32)
                    rlo = lax.bitcast_convert_type(r << 16, jnp.float32)
                    rhi = lax.bitcast_convert_type(r & himask, jnp.float32)
                    f0, f1 = 2 * k, 2 * k + 1
                    w2a = w2rows[f0 // NLANE][f0 % NLANE]
                    w2b = w2rows[f1 // NLANE][f1 % NLANE]
                    accs[f0 % 4] = accs[f0 % 4] + rlo * w2a
                    accs[f1 % 4] = accs[f1 % 4] + rhi * w2b
                acc = (accs[0] + accs[1]) + (accs[2] + accs[3])
                nzg = nzv[t, pl.ds(g * NLANE, NLANE)]
                evg = evv[t, pl.ds(g * NLANE, NLANE)]
                aug = 1.0 / (1.0 + jnp.exp(-(acc + nzg)))
                outv[t, pl.ds(g * NLANE, NLANE)] = evg * aug
                psum = psum + aug
            return psum
        return lax.fori_loop(0, CH // NLANE // 2, gbody, psum)

    for b in range(NBUF - 1):
        issue(b, pgs[b], qgs[b], sps[b], sqs[b])

    def quad(i, psum):
        t0 = NBUF * i
        for b in range(NBUF):
            t = t0 + b
            wait(t, pgs[b], qgs[b], sps[b], sqs[b])
            psum = compute(t, pgs[b], qgs[b], psum)
            b2 = (b + NBUF - 1) % NBUF

            @pl.when(t + NBUF - 1 < K)
            def _():
                issue(t + NBUF - 1, pgs[b2], qgs[b2], sps[b2], sqs[b2])

        return psum

    psum = lax.fori_loop(0, K // NBUF, quad, jnp.zeros((NLANE,), jnp.float32))
    psv[...] = psum
    pltpu.sync_copy(outv, out_hbm.at[wid])
    pltpu.sync_copy(psv, psum_hbm.at[wid])


def _make_sc_call():
    mesh = plsc.VectorSubcoreMesh(core_axis_name="c", subcore_axis_name="s")
    return pl.kernel(
        _sc_edge_body,
        mesh=mesh,
        compiler_params=pltpu.CompilerParams(
            needs_layout_passes=False,
            use_tc_tiling_on_sc=False,
        ),
        out_type=[
            jax.ShapeDtypeStruct((NW, K, CH), jnp.float32),
            jax.ShapeDtypeStruct((NW, NLANE), jnp.float32),
        ],
        scratch_types=[
            pltpu.VMEM((K, CH), jnp.int32),
            pltpu.VMEM((K, CH), jnp.int32),
            pltpu.VMEM((K, CH), jnp.float32),
            pltpu.VMEM((K, CH), jnp.float32),
            pltpu.VMEM((K, CH), jnp.float32),
            pltpu.VMEM((H,), jnp.float32),
            pltpu.VMEM((NLANE,), jnp.float32),
            *[pltpu.VMEM((CH, HPW), jnp.int32) for _ in range(2 * NBUF)],
            *[pltpu.SemaphoreType.DMA for _ in range(2 * NBUF)],
        ],
    )


def kernel(node_emb, edge_index, edge_vals, W1, b1, W2, b2):
    half = edge_index.shape[1] // 2
    src = edge_index[0, :half]
    dst = edge_index[1, :half]

    p, q = _compute_pq(node_emb, W1, b1)
    def _pack(t):
        ti = lax.bitcast_convert_type(
            t.astype(jnp.bfloat16).reshape(N, PAY, 2), jnp.int32)
        return jnp.pad(ti, ((0, 0), (0, HPW - PAY)))
    p = _pack(p)
    q = _pack(q)

    bias = 0.0 + 0.0001
    u = jax.random.uniform(jax.random.key(42), (half, 1), dtype=jnp.float32)
    eps = (bias - (1.0 - bias)) * u + (1.0 - bias)
    noise = (jnp.log(eps) - jnp.log(1.0 - eps)).squeeze(-1)
    nz = noise + b2[0]

    pad = EPAD - half
    srcp = jnp.concatenate([src, jnp.zeros((pad,), jnp.int32)]).reshape(NW, K, CH)
    dstp = jnp.concatenate([dst, jnp.zeros((pad,), jnp.int32)]).reshape(NW, K, CH)
    nzp = jnp.concatenate([nz, jnp.full((pad,), -1e30, jnp.float32)]).reshape(NW, K, CH)
    evp = jnp.concatenate([edge_vals[:half], jnp.zeros((pad,), jnp.float32)]).reshape(NW, K, CH)

    outp, psum = _make_sc_call()(p, q, srcp, dstp, nzp, evp, W2.reshape(H))

    new_vals = outp.reshape(-1)[:half]
    sym_inds = jnp.concatenate([jnp.stack([src, dst]), jnp.stack([dst, src])], axis=1)
    sym_vals = jnp.concatenate([new_vals, new_vals], axis=0)
    mean_edge_weight = jnp.sum(psum) / half
    return (sym_inds, sym_vals, mean_edge_weight)


# speedup vs baseline: 1.4829x; 1.2925x over previous
import functools

import jax
import jax.numpy as jnp
from jax import lax
from jax.experimental import pallas as pl
from jax.experimental.pallas import tpu as pltpu
from jax.experimental.pallas import tpu_sc as plsc

N = 10000
D = 128
H = 64
NW = 32
CH = 128
K = 40
EPAD = NW * K * CH
NLANE = 16
PAY = H // 2
HPW = PAY + 8


def _bf16_pack(f):
    u = lax.bitcast_convert_type(f, jnp.uint32)
    r = (u + jnp.uint32(0x7FFF) + ((u >> 16) & jnp.uint32(1))) >> 16
    packed = r[:, :PAY] | (r[:, PAY:] << 16)
    return lax.bitcast_convert_type(packed, jnp.int32)


def _pq_body(ne_ref, w1_ref, b1_ref, p_ref, q_ref):
    x = ne_ref[...]
    w1 = w1_ref[...]
    pf = lax.dot_general(x, w1[:D, :], (((1,), (0,)), ((), ())),
                         preferred_element_type=jnp.float32)
    qf = lax.dot_general(x, w1[D:, :], (((1,), (0,)), ((), ())),
                         preferred_element_type=jnp.float32) + b1_ref[...]
    p_ref[:, :PAY] = _bf16_pack(pf)
    q_ref[:, :PAY] = _bf16_pack(qf)
    p_ref[:, PAY:] = jnp.zeros_like(p_ref[:, PAY:])
    q_ref[:, PAY:] = jnp.zeros_like(q_ref[:, PAY:])


def _compute_pq(node_emb, W1, b1):
    blk = 1000
    return pl.pallas_call(
        _pq_body,
        grid=(N // blk,),
        in_specs=[
            pl.BlockSpec((blk, D), lambda i: (i, 0)),
            pl.BlockSpec((2 * D, H), lambda i: (0, 0)),
            pl.BlockSpec((1, H), lambda i: (0, 0)),
        ],
        out_specs=[
            pl.BlockSpec((blk, HPW), lambda i: (i, 0)),
            pl.BlockSpec((blk, HPW), lambda i: (i, 0)),
        ],
        out_shape=[
            jax.ShapeDtypeStruct((N, HPW), jnp.int32),
            jax.ShapeDtypeStruct((N, HPW), jnp.int32),
        ],
    )(node_emb, W1, b1.reshape(1, H))


NBUF = 4


def _sc_edge_body(p_hbm, q_hbm, src_hbm, dst_hbm, nz_hbm, ev_hbm, w2_hbm,
                  out_hbm, psum_hbm,
                  srcv, dstv, nzv, evv, outv, w2v, psv,
                  *bufs):
    pgs = bufs[0:NBUF]
    qgs = bufs[NBUF:2 * NBUF]
    sps = bufs[2 * NBUF:3 * NBUF]
    sqs = bufs[3 * NBUF:4 * NBUF]
    wid = lax.axis_index("s") * 2 + lax.axis_index("c")
    pltpu.sync_copy(src_hbm.at[wid], srcv)
    pltpu.sync_copy(dst_hbm.at[wid], dstv)
    pltpu.sync_copy(nz_hbm.at[wid], nzv)
    pltpu.sync_copy(ev_hbm.at[wid], evv)
    pltpu.sync_copy(w2_hbm, w2v)
    w2rows = [w2v[pl.ds(j * NLANE, NLANE)] for j in range(H // NLANE)]

    def issue(t, pg, qg, sp, sq):
        pltpu.make_async_copy(p_hbm.at[srcv.at[t]], pg, sp).start()
        pltpu.make_async_copy(q_hbm.at[dstv.at[t]], qg, sq).start()

    def wait(t, pg, qg, sp, sq):
        pltpu.make_async_copy(p_hbm.at[srcv.at[t]], pg, sp).wait()
        pltpu.make_async_copy(q_hbm.at[dstv.at[t]], qg, sq).wait()

    def compute(t, pg, qg, psum):
        def gbody(gg, psum):
            for half_g in range(2):
                g = gg * 2 + half_g
                rows = g * NLANE + lax.iota(jnp.int32, NLANE)
                accs = [jnp.zeros((NLANE,), jnp.float32) for _ in range(4)]
                himask = jnp.full((NLANE,), -65536, jnp.int32)
                zero_b = jnp.zeros((2 * NLANE,), jnp.bfloat16)
                for k in range(PAY):
                    kidx = jnp.full((NLANE,), k, jnp.int32)
                    pk = plsc.load_gather(pg, [rows, kidx])
                    qk = plsc.load_gather(qg, [rows, kidx])
                    pb = plsc.bitcast(pk, jnp.bfloat16)
                    qb = plsc.bitcast(qk, jnp.bfloat16)
                    r = plsc.bitcast(jnp.maximum(pb + qb, zero_b), jnp.int32)
                    rlo = lax.bitcast_convert_type(r << 16, jnp.float32)
                    rhi = lax.bitcast_convert_type(r & himask, jnp.float32)
                    f0, f1 = k, k + PAY
                    w2a = w2rows[f0 // NLANE][f0 % NLANE]
                    w2b = w2rows[f1 // NLANE][f1 % NLANE]
                    accs[k % 4] = accs[k % 4] + rlo * w2a
                    accs[(k + 2) % 4] = accs[(k + 2) % 4] + rhi * w2b
                acc = (accs[0] + accs[1]) + (accs[2] + accs[3])
                nzg = nzv[t, pl.ds(g * NLANE, NLANE)]
                evg = evv[t, pl.ds(g * NLANE, NLANE)]
                aug = 1.0 / (1.0 + jnp.exp(-(acc + nzg)))
                outv[t, pl.ds(g * NLANE, NLANE)] = evg * aug
                psum = psum + aug
            return psum
        return lax.fori_loop(0, CH // NLANE // 2, gbody, psum)

    for b in range(NBUF - 1):
        issue(b, pgs[b], qgs[b], sps[b], sqs[b])

    def quad(i, psum):
        t0 = NBUF * i
        for b in range(NBUF):
            t = t0 + b
            wait(t, pgs[b], qgs[b], sps[b], sqs[b])
            psum = compute(t, pgs[b], qgs[b], psum)
            b2 = (b + NBUF - 1) % NBUF

            @pl.when(t + NBUF - 1 < K)
            def _():
                issue(t + NBUF - 1, pgs[b2], qgs[b2], sps[b2], sqs[b2])

        return psum

    psum = lax.fori_loop(0, K // NBUF, quad, jnp.zeros((NLANE,), jnp.float32))
    psv[...] = psum
    pltpu.sync_copy(outv, out_hbm.at[wid])
    pltpu.sync_copy(psv, psum_hbm.at[wid])


def _make_sc_call():
    mesh = plsc.VectorSubcoreMesh(core_axis_name="c", subcore_axis_name="s")
    return pl.kernel(
        _sc_edge_body,
        mesh=mesh,
        compiler_params=pltpu.CompilerParams(
            needs_layout_passes=False,
            use_tc_tiling_on_sc=False,
        ),
        out_type=[
            jax.ShapeDtypeStruct((NW, K, CH), jnp.float32),
            jax.ShapeDtypeStruct((NW, NLANE), jnp.float32),
        ],
        scratch_types=[
            pltpu.VMEM((K, CH), jnp.int32),
            pltpu.VMEM((K, CH), jnp.int32),
            pltpu.VMEM((K, CH), jnp.float32),
            pltpu.VMEM((K, CH), jnp.float32),
            pltpu.VMEM((K, CH), jnp.float32),
            pltpu.VMEM((H,), jnp.float32),
            pltpu.VMEM((NLANE,), jnp.float32),
            *[pltpu.VMEM((CH, HPW), jnp.int32) for _ in range(2 * NBUF)],
            *[pltpu.SemaphoreType.DMA for _ in range(2 * NBUF)],
        ],
    )


def kernel(node_emb, edge_index, edge_vals, W1, b1, W2, b2):
    half = edge_index.shape[1] // 2
    src = edge_index[0, :half]
    dst = edge_index[1, :half]

    p, q = _compute_pq(node_emb, W1, b1)

    bias = 0.0 + 0.0001
    u = jax.random.uniform(jax.random.key(42), (half, 1), dtype=jnp.float32)
    eps = (bias - (1.0 - bias)) * u + (1.0 - bias)
    noise = (jnp.log(eps) - jnp.log(1.0 - eps)).squeeze(-1)
    nz = noise + b2[0]

    pad = EPAD - half
    srcp = jnp.concatenate([src, jnp.zeros((pad,), jnp.int32)]).reshape(NW, K, CH)
    dstp = jnp.concatenate([dst, jnp.zeros((pad,), jnp.int32)]).reshape(NW, K, CH)
    nzp = jnp.concatenate([nz, jnp.full((pad,), -1e30, jnp.float32)]).reshape(NW, K, CH)
    evp = jnp.concatenate([edge_vals[:half], jnp.zeros((pad,), jnp.float32)]).reshape(NW, K, CH)

    outp, psum = _make_sc_call()(p, q, srcp, dstp, nzp, evp, W2.reshape(H))

    new_vals = outp.reshape(-1)[:half]
    eh = edge_index[:, :half]
    sym_inds = jnp.concatenate([eh, eh[::-1]], axis=1)
    sym_vals = jnp.concatenate([new_vals, new_vals], axis=0)
    mean_edge_weight = jnp.sum(psum) / half
    return (sym_inds, sym_vals, mean_edge_weight)


# speedup vs baseline: 1.5348x; 1.0350x over previous
import functools

import jax
import jax.numpy as jnp
from jax import lax
from jax.experimental import pallas as pl
from jax.experimental.pallas import tpu as pltpu
from jax.experimental.pallas import tpu_sc as plsc

N = 10000
D = 128
H = 64
NW = 32
CH = 128
K = 40
EPAD = NW * K * CH
NLANE = 16
PAY = H // 2
HPW = PAY + 8


def _bf16_pack(f):
    u = lax.bitcast_convert_type(f, jnp.uint32)
    r = (u + jnp.uint32(0x7FFF) + ((u >> 16) & jnp.uint32(1))) >> 16
    packed = r[:, :PAY] | (r[:, PAY:] << 16)
    return lax.bitcast_convert_type(packed, jnp.int32)


def _pq_body(ne_ref, w1_ref, b1_ref, p_ref, q_ref):
    x = ne_ref[...]
    w1 = w1_ref[...]
    pf = lax.dot_general(x, w1[:D, :], (((1,), (0,)), ((), ())),
                         preferred_element_type=jnp.float32)
    qf = lax.dot_general(x, w1[D:, :], (((1,), (0,)), ((), ())),
                         preferred_element_type=jnp.float32) + b1_ref[...]
    p_ref[:, :PAY] = _bf16_pack(pf)
    q_ref[:, :PAY] = _bf16_pack(qf)
    p_ref[:, PAY:] = jnp.zeros_like(p_ref[:, PAY:])
    q_ref[:, PAY:] = jnp.zeros_like(q_ref[:, PAY:])


def _compute_pq(node_emb, W1, b1):
    blk = 1000
    return pl.pallas_call(
        _pq_body,
        grid=(N // blk,),
        in_specs=[
            pl.BlockSpec((blk, D), lambda i: (i, 0)),
            pl.BlockSpec((2 * D, H), lambda i: (0, 0)),
            pl.BlockSpec((1, H), lambda i: (0, 0)),
        ],
        out_specs=[
            pl.BlockSpec((blk, HPW), lambda i: (i, 0)),
            pl.BlockSpec((blk, HPW), lambda i: (i, 0)),
        ],
        out_shape=[
            jax.ShapeDtypeStruct((N, HPW), jnp.int32),
            jax.ShapeDtypeStruct((N, HPW), jnp.int32),
        ],
    )(node_emb, W1, b1.reshape(1, H))


NBUF = 4


def _sc_edge_body(p_hbm, q_hbm, src_hbm, dst_hbm, nz_hbm, ev_hbm, w2_hbm,
                  out_hbm, psum_hbm,
                  srcv, dstv, nzv, evv, outv, w2v, psv,
                  *bufs):
    pgs = bufs[0:NBUF]
    qgs = bufs[NBUF:2 * NBUF]
    sps = bufs[2 * NBUF:3 * NBUF]
    sqs = bufs[3 * NBUF:4 * NBUF]
    p_sp = bufs[4 * NBUF]
    sid = lax.axis_index("s")
    wid = sid * 2 + lax.axis_index("c")
    nrows = N // 16
    pltpu.sync_copy(p_hbm.at[pl.ds(sid * nrows, nrows)],
                    p_sp.at[pl.ds(sid * nrows, nrows)])
    plsc.subcore_barrier()
    pltpu.sync_copy(src_hbm.at[wid], srcv)
    pltpu.sync_copy(dst_hbm.at[wid], dstv)
    pltpu.sync_copy(nz_hbm.at[wid], nzv)
    pltpu.sync_copy(ev_hbm.at[wid], evv)
    pltpu.sync_copy(w2_hbm, w2v)
    w2rows = [w2v[pl.ds(j * NLANE, NLANE)] for j in range(H // NLANE)]

    def issue(t, pg, qg, sp, sq):
        pltpu.make_async_copy(p_sp.at[srcv.at[t]], pg, sp).start()
        pltpu.make_async_copy(q_hbm.at[dstv.at[t]], qg, sq).start()

    def wait(t, pg, qg, sp, sq):
        pltpu.make_async_copy(p_sp.at[srcv.at[t]], pg, sp).wait()
        pltpu.make_async_copy(q_hbm.at[dstv.at[t]], qg, sq).wait()

    def compute(t, pg, qg, psum):
        def gbody(gg, psum):
            for half_g in range(2):
                g = gg * 2 + half_g
                rows = g * NLANE + lax.iota(jnp.int32, NLANE)
                accs = [jnp.zeros((NLANE,), jnp.float32) for _ in range(4)]
                himask = jnp.full((NLANE,), -65536, jnp.int32)
                zero_b = jnp.zeros((2 * NLANE,), jnp.bfloat16)
                for k in range(PAY):
                    kidx = jnp.full((NLANE,), k, jnp.int32)
                    pk = plsc.load_gather(pg, [rows, kidx])
                    qk = plsc.load_gather(qg, [rows, kidx])
                    pb = plsc.bitcast(pk, jnp.bfloat16)
                    qb = plsc.bitcast(qk, jnp.bfloat16)
                    r = plsc.bitcast(jnp.maximum(pb + qb, zero_b), jnp.int32)
                    rlo = lax.bitcast_convert_type(r << 16, jnp.float32)
                    rhi = lax.bitcast_convert_type(r & himask, jnp.float32)
                    f0, f1 = k, k + PAY
                    w2a = w2rows[f0 // NLANE][f0 % NLANE]
                    w2b = w2rows[f1 // NLANE][f1 % NLANE]
                    accs[k % 4] = accs[k % 4] + rlo * w2a
                    accs[(k + 2) % 4] = accs[(k + 2) % 4] + rhi * w2b
                acc = (accs[0] + accs[1]) + (accs[2] + accs[3])
                nzg = nzv[t, pl.ds(g * NLANE, NLANE)]
                evg = evv[t, pl.ds(g * NLANE, NLANE)]
                aug = 1.0 / (1.0 + jnp.exp(-(acc + nzg)))
                outv[t, pl.ds(g * NLANE, NLANE)] = evg * aug
                psum = psum + aug
            return psum
        return lax.fori_loop(0, CH // NLANE // 2, gbody, psum)

    for b in range(NBUF - 1):
        issue(b, pgs[b], qgs[b], sps[b], sqs[b])

    def quad(i, psum):
        t0 = NBUF * i
        for b in range(NBUF):
            t = t0 + b
            wait(t, pgs[b], qgs[b], sps[b], sqs[b])
            psum = compute(t, pgs[b], qgs[b], psum)
            b2 = (b + NBUF - 1) % NBUF

            @pl.when(t + NBUF - 1 < K)
            def _():
                issue(t + NBUF - 1, pgs[b2], qgs[b2], sps[b2], sqs[b2])

        return psum

    psum = lax.fori_loop(0, K // NBUF, quad, jnp.zeros((NLANE,), jnp.float32))
    psv[...] = psum
    pltpu.sync_copy(outv, out_hbm.at[wid])
    pltpu.sync_copy(psv, psum_hbm.at[wid])


def _make_sc_call():
    mesh = plsc.VectorSubcoreMesh(core_axis_name="c", subcore_axis_name="s")
    return pl.kernel(
        _sc_edge_body,
        mesh=mesh,
        compiler_params=pltpu.CompilerParams(
            needs_layout_passes=False,
            use_tc_tiling_on_sc=False,
        ),
        out_type=[
            jax.ShapeDtypeStruct((NW, K, CH), jnp.float32),
            jax.ShapeDtypeStruct((NW, NLANE), jnp.float32),
        ],
        scratch_types=[
            pltpu.VMEM((K, CH), jnp.int32),
            pltpu.VMEM((K, CH), jnp.int32),
            pltpu.VMEM((K, CH), jnp.float32),
            pltpu.VMEM((K, CH), jnp.float32),
            pltpu.VMEM((K, CH), jnp.float32),
            pltpu.VMEM((H,), jnp.float32),
            pltpu.VMEM((NLANE,), jnp.float32),
            *[pltpu.VMEM((CH, HPW), jnp.int32) for _ in range(2 * NBUF)],
            *[pltpu.SemaphoreType.DMA for _ in range(2 * NBUF)],
            pltpu.VMEM_SHARED((N, HPW), jnp.int32),
        ],
    )


def kernel(node_emb, edge_index, edge_vals, W1, b1, W2, b2):
    half = edge_index.shape[1] // 2
    src = edge_index[0, :half]
    dst = edge_index[1, :half]

    p, q = _compute_pq(node_emb, W1, b1)

    bias = 0.0 + 0.0001
    u = jax.random.uniform(jax.random.key(42), (half, 1), dtype=jnp.float32)
    eps = (bias - (1.0 - bias)) * u + (1.0 - bias)
    noise = (jnp.log(eps) - jnp.log(1.0 - eps)).squeeze(-1)
    nz = noise + b2[0]

    pad = EPAD - half
    srcp = jnp.concatenate([src, jnp.zeros((pad,), jnp.int32)]).reshape(NW, K, CH)
    dstp = jnp.concatenate([dst, jnp.zeros((pad,), jnp.int32)]).reshape(NW, K, CH)
    nzp = jnp.concatenate([nz, jnp.full((pad,), -1e30, jnp.float32)]).reshape(NW, K, CH)
    evp = jnp.concatenate([edge_vals[:half], jnp.zeros((pad,), jnp.float32)]).reshape(NW, K, CH)

    outp, psum = _make_sc_call()(p, q, srcp, dstp, nzp, evp, W2.reshape(H))

    new_vals = outp.reshape(-1)[:half]
    eh = edge_index[:, :half]
    sym_inds = jnp.concatenate([eh, eh[::-1]], axis=1)
    sym_vals = jnp.concatenate([new_vals, new_vals], axis=0)
    mean_edge_weight = jnp.sum(psum) / half
    return (sym_inds, sym_vals, mean_edge_weight)


# speedup vs baseline: 1.7428x; 1.1356x over previous
import functools

import jax
import jax.numpy as jnp
from jax import lax
from jax.experimental import pallas as pl
from jax.experimental.pallas import tpu as pltpu
from jax.experimental.pallas import tpu_sc as plsc

N = 10000
D = 128
H = 64
NW = 32
CH = 128
K = 40
EPAD = NW * K * CH
NLANE = 16
PAY = H // 2
HPW = PAY + 8


def _bf16_pack(f):
    u = lax.bitcast_convert_type(f, jnp.uint32)
    r = (u + jnp.uint32(0x7FFF) + ((u >> 16) & jnp.uint32(1))) >> 16
    packed = r[:, :PAY] | (r[:, PAY:] << 16)
    return lax.bitcast_convert_type(packed, jnp.int32)


def _pq_body(ne_ref, w1_ref, b1_ref, p_ref, q_ref):
    x = ne_ref[...]
    w1 = w1_ref[...]
    pf = lax.dot_general(x, w1[:D, :], (((1,), (0,)), ((), ())),
                         preferred_element_type=jnp.float32)
    qf = lax.dot_general(x, w1[D:, :], (((1,), (0,)), ((), ())),
                         preferred_element_type=jnp.float32) + b1_ref[...]
    p_ref[:, :PAY] = _bf16_pack(pf)
    q_ref[:, :PAY] = _bf16_pack(qf)
    p_ref[:, PAY:] = jnp.zeros_like(p_ref[:, PAY:])
    q_ref[:, PAY:] = jnp.zeros_like(q_ref[:, PAY:])


def _compute_pq(node_emb, W1, b1):
    blk = 1000
    return pl.pallas_call(
        _pq_body,
        grid=(N // blk,),
        in_specs=[
            pl.BlockSpec((blk, D), lambda i: (i, 0)),
            pl.BlockSpec((2 * D, H), lambda i: (0, 0)),
            pl.BlockSpec((1, H), lambda i: (0, 0)),
        ],
        out_specs=[
            pl.BlockSpec((blk, HPW), lambda i: (i, 0)),
            pl.BlockSpec((blk, HPW), lambda i: (i, 0)),
        ],
        out_shape=[
            jax.ShapeDtypeStruct((N, HPW), jnp.int32),
            jax.ShapeDtypeStruct((N, HPW), jnp.int32),
        ],
    )(node_emb, W1, b1.reshape(1, H))


NBUF = 4


def _sc_edge_body(p_hbm, q_hbm, src_hbm, dst_hbm, nz_hbm, ev_hbm, w2_hbm,
                  out_hbm, psum_hbm,
                  srcv, dstv, nzv, evv, outv, w2v, psv,
                  *bufs):
    pgs = bufs[0:NBUF]
    qgs = bufs[NBUF:2 * NBUF]
    sps = bufs[2 * NBUF:3 * NBUF]
    sqs = bufs[3 * NBUF:4 * NBUF]
    p_sp = bufs[4 * NBUF]
    q_sp = bufs[4 * NBUF + 1]
    sid = lax.axis_index("s")
    wid = sid * 2 + lax.axis_index("c")
    nrows = N // 16
    pltpu.sync_copy(p_hbm.at[pl.ds(sid * nrows, nrows)],
                    p_sp.at[pl.ds(sid * nrows, nrows)])
    pltpu.sync_copy(q_hbm.at[pl.ds(sid * nrows, nrows)],
                    q_sp.at[pl.ds(sid * nrows, nrows)])
    plsc.subcore_barrier()
    pltpu.sync_copy(src_hbm.at[wid], srcv)
    pltpu.sync_copy(dst_hbm.at[wid], dstv)
    pltpu.sync_copy(nz_hbm.at[wid], nzv)
    pltpu.sync_copy(ev_hbm.at[wid], evv)
    pltpu.sync_copy(w2_hbm, w2v)
    w2rows = [w2v[pl.ds(j * NLANE, NLANE)] for j in range(H // NLANE)]

    def issue(t, pg, qg, sp, sq):
        pltpu.make_async_copy(p_sp.at[srcv.at[t]], pg, sp).start()
        pltpu.make_async_copy(q_sp.at[dstv.at[t]], qg, sq).start()

    def wait(t, pg, qg, sp, sq):
        pltpu.make_async_copy(p_sp.at[srcv.at[t]], pg, sp).wait()
        pltpu.make_async_copy(q_sp.at[dstv.at[t]], qg, sq).wait()

    def compute(t, pg, qg, psum):
        def gbody(gg, psum):
            for half_g in range(2):
                g = gg * 2 + half_g
                rows = g * NLANE + lax.iota(jnp.int32, NLANE)
                accs = [jnp.zeros((NLANE,), jnp.float32) for _ in range(4)]
                himask = jnp.full((NLANE,), -65536, jnp.int32)
                zero_b = jnp.zeros((2 * NLANE,), jnp.bfloat16)
                for k in range(PAY):
                    kidx = jnp.full((NLANE,), k, jnp.int32)
                    pk = plsc.load_gather(pg, [rows, kidx])
                    qk = plsc.load_gather(qg, [rows, kidx])
                    pb = plsc.bitcast(pk, jnp.bfloat16)
                    qb = plsc.bitcast(qk, jnp.bfloat16)
                    r = plsc.bitcast(jnp.maximum(pb + qb, zero_b), jnp.int32)
                    rlo = lax.bitcast_convert_type(r << 16, jnp.float32)
                    rhi = lax.bitcast_convert_type(r & himask, jnp.float32)
                    f0, f1 = k, k + PAY
                    w2a = w2rows[f0 // NLANE][f0 % NLANE]
                    w2b = w2rows[f1 // NLANE][f1 % NLANE]
                    accs[k % 4] = accs[k % 4] + rlo * w2a
                    accs[(k + 2) % 4] = accs[(k + 2) % 4] + rhi * w2b
                acc = (accs[0] + accs[1]) + (accs[2] + accs[3])
                nzg = nzv[t, pl.ds(g * NLANE, NLANE)]
                evg = evv[t, pl.ds(g * NLANE, NLANE)]
                aug = 1.0 / (1.0 + jnp.exp(-(acc + nzg)))
                outv[t, pl.ds(g * NLANE, NLANE)] = evg * aug
                psum = psum + aug
            return psum
        return lax.fori_loop(0, CH // NLANE // 2, gbody, psum)

    for b in range(NBUF - 1):
        issue(b, pgs[b], qgs[b], sps[b], sqs[b])

    def quad(i, psum):
        t0 = NBUF * i
        for b in range(NBUF):
            t = t0 + b
            wait(t, pgs[b], qgs[b], sps[b], sqs[b])
            psum = compute(t, pgs[b], qgs[b], psum)
            b2 = (b + NBUF - 1) % NBUF

            @pl.when(t + NBUF - 1 < K)
            def _():
                issue(t + NBUF - 1, pgs[b2], qgs[b2], sps[b2], sqs[b2])

        return psum

    psum = lax.fori_loop(0, K // NBUF, quad, jnp.zeros((NLANE,), jnp.float32))
    psv[...] = psum
    pltpu.sync_copy(outv, out_hbm.at[wid])
    pltpu.sync_copy(psv, psum_hbm.at[wid])


def _make_sc_call():
    mesh = plsc.VectorSubcoreMesh(core_axis_name="c", subcore_axis_name="s")
    return pl.kernel(
        _sc_edge_body,
        mesh=mesh,
        compiler_params=pltpu.CompilerParams(
            needs_layout_passes=False,
            use_tc_tiling_on_sc=False,
        ),
        out_type=[
            jax.ShapeDtypeStruct((NW, K, CH), jnp.float32),
            jax.ShapeDtypeStruct((NW, NLANE), jnp.float32),
        ],
        scratch_types=[
            pltpu.VMEM((K, CH), jnp.int32),
            pltpu.VMEM((K, CH), jnp.int32),
            pltpu.VMEM((K, CH), jnp.float32),
            pltpu.VMEM((K, CH), jnp.float32),
            pltpu.VMEM((K, CH), jnp.float32),
            pltpu.VMEM((H,), jnp.float32),
            pltpu.VMEM((NLANE,), jnp.float32),
            *[pltpu.VMEM((CH, HPW), jnp.int32) for _ in range(2 * NBUF)],
            *[pltpu.SemaphoreType.DMA for _ in range(2 * NBUF)],
            pltpu.VMEM_SHARED((N, HPW), jnp.int32),
            pltpu.VMEM_SHARED((N, HPW), jnp.int32),
        ],
    )


def kernel(node_emb, edge_index, edge_vals, W1, b1, W2, b2):
    half = edge_index.shape[1] // 2
    src = edge_index[0, :half]
    dst = edge_index[1, :half]

    p, q = _compute_pq(node_emb, W1, b1)

    bias = 0.0 + 0.0001
    u = jax.random.uniform(jax.random.key(42), (half, 1), dtype=jnp.float32)
    eps = (bias - (1.0 - bias)) * u + (1.0 - bias)
    noise = (jnp.log(eps) - jnp.log(1.0 - eps)).squeeze(-1)
    nz = noise + b2[0]

    pad = EPAD - half
    srcp = jnp.concatenate([src, jnp.zeros((pad,), jnp.int32)]).reshape(NW, K, CH)
    dstp = jnp.concatenate([dst, jnp.zeros((pad,), jnp.int32)]).reshape(NW, K, CH)
    nzp = jnp.concatenate([nz, jnp.full((pad,), -1e30, jnp.float32)]).reshape(NW, K, CH)
    evp = jnp.concatenate([edge_vals[:half], jnp.zeros((pad,), jnp.float32)]).reshape(NW, K, CH)

    outp, psum = _make_sc_call()(p, q, srcp, dstp, nzp, evp, W2.reshape(H))

    new_vals = outp.reshape(-1)[:half]
    eh = edge_index[:, :half]
    sym_inds = jnp.concatenate([eh, eh[::-1]], axis=1)
    sym_vals = jnp.concatenate([new_vals, new_vals], axis=0)
    mean_edge_weight = jnp.sum(psum) / half
    return (sym_inds, sym_vals, mean_edge_weight)
